# Initial kernel scaffold; baseline (speedup 1.0000x reference)
import jax
import jax.numpy as jnp
from jax.experimental import pallas as pl


def kernel(xyz, features, W1, b1, g1, beta1, W2, b2, g2, beta2, W3, b3, g3, beta3):
    raise NotImplementedError("write your pallas kernel here")


# speedup vs baseline: 13.1456x; 13.1456x over previous
import functools

import jax
import jax.numpy as jnp
from jax import lax
from jax.experimental import pallas as pl
from jax.experimental.pallas import tpu as pltpu
from jax.experimental.pallas import tpu_sc as plsc

B = 8
N = 4096
S = 512
K = 32
R2 = 0.2 * 0.2
C0 = 67
C1, C2, C3 = 64, 128, 256
NTOT = B * S * K
EPS = 1e-5

NC = 2
NS = 16
NW = NC * NS
ROWS_PER_W = (B * S * K) // NW
GCHUNK = 128


def _fps_body(x_ref, y_ref, z_ref, nx_ref, ny_ref, nz_ref):
    x = x_ref[...]
    y = y_ref[...]
    z = z_ref[...]
    lane = lax.broadcasted_iota(jnp.int32, (B, N), 1)
    lane_s = lax.broadcasted_iota(jnp.int32, (B, S), 1)

    def step(i, carry):
        dist, far, nx, ny, nz = carry
        sel = lane == far
        cx = jnp.sum(jnp.where(sel, x, 0.0), axis=1, keepdims=True)
        cy = jnp.sum(jnp.where(sel, y, 0.0), axis=1, keepdims=True)
        cz = jnp.sum(jnp.where(sel, z, 0.0), axis=1, keepdims=True)
        hit = lane_s == i
        nx = jnp.where(hit, cx, nx)
        ny = jnp.where(hit, cy, ny)
        nz = jnp.where(hit, cz, nz)
        d = (x - cx) ** 2 + (y - cy) ** 2 + (z - cz) ** 2
        dist = jnp.minimum(dist, d)
        m = jnp.max(dist, axis=1, keepdims=True)
        far = jnp.min(jnp.where(dist == m, lane, N), axis=1, keepdims=True)
        return dist, far, nx, ny, nz

    dist0 = jnp.full((B, N), 1e10, jnp.float32)
    far0 = jnp.zeros((B, 1), jnp.int32)
    zs = jnp.zeros((B, S), jnp.float32)
    _, _, nx, ny, nz = lax.fori_loop(0, S, step, (dist0, far0, zs, zs, zs))
    nx_ref[...] = nx
    ny_ref[...] = ny
    nz_ref[...] = nz


_fps = pl.pallas_call(
    _fps_body,
    out_shape=[jax.ShapeDtypeStruct((B, S), jnp.float32)] * 3,
)


def _bq_body(nxyz_ref, xt_ref, out_ref):
    b = pl.program_id(0)
    nxb = nxyz_ref[0]
    xtb = xt_ref[0]
    mm = jnp.dot(nxb, xtb)
    ns2 = jnp.sum(nxb * nxb, axis=1, keepdims=True)
    p2 = jnp.sum(xtb * xtb, axis=0, keepdims=True)
    d = -2.0 * mm + ns2 + p2
    mask = (d <= R2).astype(jnp.int32)

    rank = mask
    sh = 1
    while sh < N:
        shifted = jnp.concatenate(
            [jnp.zeros((S, sh), jnp.int32), rank[:, : N - sh]], axis=1)
        rank = rank + shifted
        sh *= 2
    cnt = rank[:, N - 1:N]

    cols = []
    j0 = None
    for k in range(K):
        jk = jnp.sum((rank <= k).astype(jnp.int32), axis=1, keepdims=True)
        if k == 0:
            j0 = jk
            cols.append(jk)
        else:
            cols.append(jnp.where(cnt > k, jk, j0))
    out_ref[0] = jnp.concatenate(cols, axis=1) + b * N


_bq = pl.pallas_call(
    _bq_body,
    grid=(B,),
    in_specs=[
        pl.BlockSpec((1, S, 3), lambda b: (b, 0, 0)),
        pl.BlockSpec((1, 3, N), lambda b: (b, 0, 0)),
    ],
    out_specs=pl.BlockSpec((1, S, K), lambda b: (b, 0, 0)),
    out_shape=jax.ShapeDtypeStruct((B, S, K), jnp.int32),
)


def _fw_body(xyz_ref, ft_ref, w1xt_ref, w1ft_ref, fw_ref):
    xyzb = xyz_ref[0]
    ftb = ft_ref[0]
    fw_ref[0] = (
        jnp.dot(ftb, w1ft_ref[...], preferred_element_type=jnp.float32)
        + jnp.dot(xyzb, w1xt_ref[...], preferred_element_type=jnp.float32)
    )


_fw = pl.pallas_call(
    _fw_body,
    grid=(B,),
    in_specs=[
        pl.BlockSpec((1, N, 3), lambda b: (b, 0, 0)),
        pl.BlockSpec((1, N, C1), lambda b: (b, 0, 0)),
        pl.BlockSpec((3, C1), lambda b: (0, 0)),
        pl.BlockSpec((C1, C1), lambda b: (0, 0)),
    ],
    out_specs=pl.BlockSpec((1, N, C1), lambda b: (b, 0, 0)),
    out_shape=jax.ShapeDtypeStruct((B, N, C1), jnp.float32),
)


def _sc_gather_body(table_hbm, idx_hbm, out_hbm, idx_v, rows_v, sem):
    wid = lax.axis_index("s") * NC + lax.axis_index("c")
    base = wid * ROWS_PER_W

    def body(t, _):
        off = base + t * GCHUNK
        pltpu.sync_copy(idx_hbm.at[pl.ds(off, GCHUNK)], idx_v)
        pltpu.async_copy(table_hbm.at[idx_v], rows_v, sem).wait()
        pltpu.sync_copy(rows_v, out_hbm.at[pl.ds(off, GCHUNK)])
        return 0

    lax.fori_loop(0, ROWS_PER_W // GCHUNK, body, 0)


@functools.cache
def _sc_gather_call():
    mesh = plsc.VectorSubcoreMesh(
        core_axis_name="c", subcore_axis_name="s",
        num_cores=NC, num_subcores=NS)
    return pl.kernel(
        _sc_gather_body,
        mesh=mesh,
        out_type=jax.ShapeDtypeStruct((NTOT, C1), jnp.float32),
        scratch_types=[
            pltpu.VMEM((GCHUNK,), jnp.int32),
            pltpu.VMEM((GCHUNK, C1), jnp.float32),
            pltpu.SemaphoreType.DMA,
        ],
        compiler_params=pltpu.CompilerParams(use_tc_tiling_on_sc=False),
    )


def _sc_gather(fw, idx):
    return _sc_gather_call()(fw, idx)


NCHUNK = 32
CH = NTOT // NCHUNK
SCH = CH // K


def _corr_block(nx_ref, w1xt_ref, b1_ref):
    corr = (
        jnp.dot(nx_ref[0], w1xt_ref[...], preferred_element_type=jnp.float32)
        - b1_ref[...]
    )
    corr = jnp.broadcast_to(corr.reshape(SCH, 1, C1), (SCH, K, C1))
    return corr.reshape(CH, C1)


def _accum_stats(i, st_ref, y):
    @pl.when(i == 0)
    def _():
        st_ref[...] = jnp.zeros_like(st_ref)

    st_ref[0:1, :] += jnp.sum(y, axis=0, keepdims=True)
    st_ref[1:2, :] += jnp.sum(y * y, axis=0, keepdims=True)


def _pa_body(g_ref, nx_ref, w1xt_ref, b1_ref, st_ref):
    i = pl.program_id(0)
    y1 = g_ref[...] - _corr_block(nx_ref, w1xt_ref, b1_ref)
    _accum_stats(i, st_ref, y1)


_pa = pl.pallas_call(
    _pa_body,
    grid=(NCHUNK,),
    in_specs=[
        pl.BlockSpec((CH, C1), lambda i: (i, 0)),
        pl.BlockSpec((1, SCH, 3), lambda i: (0, i, 0)),
        pl.BlockSpec((3, C1), lambda i: (0, 0)),
        pl.BlockSpec((1, C1), lambda i: (0, 0)),
    ],
    out_specs=pl.BlockSpec((8, C1), lambda i: (0, 0)),
    out_shape=jax.ShapeDtypeStruct((8, C1), jnp.float32),
)


def _pb_body(g_ref, nx_ref, w1xt_ref, b1_ref, a1_ref, c1_ref, w2t_ref,
             b2_ref, y2_ref, st_ref):
    i = pl.program_id(0)
    y1 = g_ref[...] - _corr_block(nx_ref, w1xt_ref, b1_ref)
    x1 = jnp.maximum(y1 * a1_ref[...] + c1_ref[...], 0.0)
    y2 = jnp.dot(x1, w2t_ref[...], preferred_element_type=jnp.float32) + b2_ref[...]
    y2_ref[...] = y2
    _accum_stats(i, st_ref, y2)


_pb = pl.pallas_call(
    _pb_body,
    grid=(NCHUNK,),
    in_specs=[
        pl.BlockSpec((CH, C1), lambda i: (i, 0)),
        pl.BlockSpec((1, SCH, 3), lambda i: (0, i, 0)),
        pl.BlockSpec((3, C1), lambda i: (0, 0)),
        pl.BlockSpec((1, C1), lambda i: (0, 0)),
        pl.BlockSpec((1, C1), lambda i: (0, 0)),
        pl.BlockSpec((1, C1), lambda i: (0, 0)),
        pl.BlockSpec((C1, C2), lambda i: (0, 0)),
        pl.BlockSpec((1, C2), lambda i: (0, 0)),
    ],
    out_specs=[
        pl.BlockSpec((CH, C2), lambda i: (i, 0)),
        pl.BlockSpec((8, C2), lambda i: (0, 0)),
    ],
    out_shape=[
        jax.ShapeDtypeStruct((NTOT, C2), jnp.float32),
        jax.ShapeDtypeStruct((8, C2), jnp.float32),
    ],
)


def _pc_body(y2_ref, a2_ref, c2_ref, w3t_ref, b3_ref, mx_ref, mn_ref, st_ref):
    i = pl.program_id(0)
    x2 = jnp.maximum(y2_ref[...] * a2_ref[...] + c2_ref[...], 0.0)
    y3 = jnp.dot(x2, w3t_ref[...], preferred_element_type=jnp.float32) + b3_ref[...]
    _accum_stats(i, st_ref, y3)
    y3r = y3.reshape(SCH, K, C3)
    mx_ref[...] = jnp.max(y3r, axis=1)
    mn_ref[...] = jnp.min(y3r, axis=1)


_pc = pl.pallas_call(
    _pc_body,
    grid=(NCHUNK,),
    in_specs=[
        pl.BlockSpec((CH, C2), lambda i: (i, 0)),
        pl.BlockSpec((1, C2), lambda i: (0, 0)),
        pl.BlockSpec((1, C2), lambda i: (0, 0)),
        pl.BlockSpec((C2, C3), lambda i: (0, 0)),
        pl.BlockSpec((1, C3), lambda i: (0, 0)),
    ],
    out_specs=[
        pl.BlockSpec((SCH, C3), lambda i: (i, 0)),
        pl.BlockSpec((SCH, C3), lambda i: (i, 0)),
        pl.BlockSpec((8, C3), lambda i: (0, 0)),
    ],
    out_shape=[
        jax.ShapeDtypeStruct((B * S, C3), jnp.float32),
        jax.ShapeDtypeStruct((B * S, C3), jnp.float32),
        jax.ShapeDtypeStruct((8, C3), jnp.float32),
    ],
)


def _ep_body(mx_ref, mn_ref, a3_ref, c3_ref, out_ref):
    hi = mx_ref[...] * a3_ref[...] + c3_ref[...]
    lo = mn_ref[...] * a3_ref[...] + c3_ref[...]
    out_ref[...] = jnp.maximum(jnp.maximum(hi, lo), 0.0)


_ep = pl.pallas_call(
    _ep_body,
    out_shape=jax.ShapeDtypeStruct((B * S, C3), jnp.float32),
)


def _bn_coefs(st, g, beta):
    mu = st[0] / NTOT
    var = st[1] / NTOT - mu * mu
    a = g / jnp.sqrt(var + EPS)
    c = beta - mu * a
    return a.reshape(1, -1), c.reshape(1, -1)


def kernel(xyz, features, W1, b1, g1, beta1, W2, b2, g2, beta2,
           W3, b3, g3, beta3):
    x8 = xyz[:, :, 0]
    y8 = xyz[:, :, 1]
    z8 = xyz[:, :, 2]
    nx8, ny8, nz8 = _fps(x8, y8, z8)
    new_xyz = jnp.stack([nx8, ny8, nz8], axis=-1)

    gidx = _bq(new_xyz, jnp.swapaxes(xyz, 1, 2))
    flat_idx = gidx.reshape(-1)

    w1t = W1.T
    w1xt = w1t[:3]
    w1ft = w1t[3:]
    ft = jnp.swapaxes(features, 1, 2)
    fw = _fw(xyz, ft, w1xt, w1ft).reshape(B * N, C1)

    g = _sc_gather(fw, flat_idx)

    nxc = new_xyz.reshape(1, B * S, 3)
    b1r = b1.reshape(1, -1)
    st1 = _pa(g, nxc, w1xt, b1r)
    a1, c1 = _bn_coefs(st1, g1, beta1)

    y2, st2 = _pb(g, nxc, w1xt, b1r, a1, c1, W2.T, b2.reshape(1, -1))
    a2, c2 = _bn_coefs(st2, g2, beta2)

    mx, mn, st3 = _pc(y2, a2, c2, W3.T, b3.reshape(1, -1))
    a3, c3 = _bn_coefs(st3, g3, beta3)

    out = _ep(mx, mn, a3, c3)
    new_features = out.reshape(B, S, C3).transpose(0, 2, 1)
    return new_xyz, new_features
